# baseline (speedup 1.0000x reference)
import functools

import jax
import jax.numpy as jnp
from jax import lax
from jax.experimental import pallas as pl
from jax.experimental.pallas import tpu as pltpu

MARGIN_ = 1.0
EPS_ = 1e-6
CB_ = 4096
KAUG_ = 128


def _dist_body(a_aug_ref, labels_ref, target_ref, minacc_ref, posacc_ref,
               *, n_valid):
    i = pl.program_id(0)
    B = a_aug_ref.shape[0]
    D = target_ref.shape[1]

    rows = i * CB_ + lax.broadcasted_iota(jnp.int32, (CB_, 1), 0)
    pad = rows >= n_valid
    t = jnp.where(pad, 0.0, target_ref[...] - EPS_)
    t_sq = jnp.sum(t * t, axis=1, keepdims=True)
    t_sq = jnp.where(pad, 3e38, t_sq)
    t_aug = jnp.concatenate(
        [t, t_sq, jnp.zeros((CB_, KAUG_ - D - 1), jnp.float32)], axis=1)

    s = lax.dot_general(a_aug_ref[...], t_aug, (((1,), (1,)), ((), ())),
                        preferred_element_type=jnp.float32)

    @pl.when(i == 0)
    def _init():
        minacc_ref[...] = jnp.full_like(minacc_ref, jnp.inf)
        posacc_ref[...] = jnp.zeros_like(posacc_ref)

    lbl_s = labels_ref[...] - i * CB_
    lane = lax.broadcasted_iota(jnp.int32, (B, 128), 1)
    m = minacc_ref[...]
    p = posacc_ref[...]
    for k in range(CB_ // 128):
        sk = s[:, k * 128:(k + 1) * 128]
        own = (lane + k * 128) == lbl_s
        m = jnp.minimum(m, jnp.where(own, jnp.inf, sk))
        p = p + jnp.where(own, sk, 0.0)
    minacc_ref[...] = m
    posacc_ref[...] = p


def _final_body(minacc_ref, posacc_ref, inputs_ref, out_ref):
    a = inputs_ref[...]
    a_sq = jnp.sum(a * a, axis=1, keepdims=True)
    d_an = jnp.sqrt(jnp.clip(
        a_sq + jnp.min(minacc_ref[...], axis=1, keepdims=True), 1e-12))
    d_ap = jnp.sqrt(jnp.clip(
        a_sq + jnp.sum(posacc_ref[...], axis=1, keepdims=True), 1e-12))
    per = jnp.maximum(d_ap - d_an + MARGIN_, 0.0)
    out_ref[0, 0] = jnp.sum(per) / a.shape[0]


def kernel(inputs, labels, target):
    B, D = inputs.shape
    C = target.shape[0]
    nblocks = (C + CB_ - 1) // CB_

    a_aug = jnp.concatenate(
        [-2.0 * inputs,
         jnp.ones((B, 1), jnp.float32),
         jnp.zeros((B, KAUG_ - D - 1), jnp.float32)], axis=1)
    labels2 = labels.reshape(B, 1)

    minacc, posacc = pl.pallas_call(
        functools.partial(_dist_body, n_valid=C),
        grid=(nblocks,),
        in_specs=[
            pl.BlockSpec((B, KAUG_), lambda i: (0, 0)),
            pl.BlockSpec((B, 1), lambda i: (0, 0)),
            pl.BlockSpec((CB_, D), lambda i: (i, 0)),
        ],
        out_specs=[
            pl.BlockSpec((B, 128), lambda i: (0, 0)),
            pl.BlockSpec((B, 128), lambda i: (0, 0)),
        ],
        out_shape=[
            jax.ShapeDtypeStruct((B, 128), jnp.float32),
            jax.ShapeDtypeStruct((B, 128), jnp.float32),
        ],
        compiler_params=pltpu.CompilerParams(
            dimension_semantics=("arbitrary",)),
    )(a_aug, labels2, target)

    out = pl.pallas_call(
        _final_body,
        out_specs=pl.BlockSpec(memory_space=pltpu.SMEM),
        out_shape=jax.ShapeDtypeStruct((1, 1), jnp.float32),
    )(minacc, posacc, inputs)
    return out[0, 0]


# speedup vs baseline: 1.3544x; 1.3544x over previous
import functools

import jax
import jax.numpy as jnp
from jax import lax
from jax.experimental import pallas as pl
from jax.experimental.pallas import tpu as pltpu

MARGIN_ = 1.0
EPS_ = 1e-6
CB_ = 1024
KAUG_ = 128


def _dist_body(a_aug_ref, labels_ref, target_ref, minacc_ref, posacc_ref,
               *, n_valid):
    i = pl.program_id(0)
    B = a_aug_ref.shape[0]
    D = target_ref.shape[1]

    rows = i * CB_ + lax.broadcasted_iota(jnp.int32, (CB_, 1), 0)
    pad = rows >= n_valid
    t = jnp.where(pad, 0.0, target_ref[...] - EPS_)
    t_sq = jnp.sum(t * t, axis=1, keepdims=True)
    t_sq = jnp.where(pad, 3e38, t_sq)
    t_aug = jnp.concatenate(
        [t, t_sq, jnp.zeros((CB_, KAUG_ - D - 1), jnp.float32)], axis=1)

    s = lax.dot_general(a_aug_ref[...], t_aug, (((1,), (1,)), ((), ())),
                        preferred_element_type=jnp.float32)

    @pl.when(i == 0)
    def _init():
        minacc_ref[...] = jnp.full_like(minacc_ref, jnp.inf)
        posacc_ref[...] = jnp.zeros_like(posacc_ref)

    lbl_s = labels_ref[...] - i * CB_
    lane = lax.broadcasted_iota(jnp.int32, (B, 128), 1)
    m = minacc_ref[...]
    p = posacc_ref[...]
    for k in range(CB_ // 128):
        sk = s[:, k * 128:(k + 1) * 128]
        own = (lane + k * 128) == lbl_s
        m = jnp.minimum(m, jnp.where(own, jnp.inf, sk))
        p = p + jnp.where(own, sk, 0.0)
    minacc_ref[...] = m
    posacc_ref[...] = p


def _final_body(minacc_ref, posacc_ref, inputs_ref, out_ref):
    a = inputs_ref[...]
    a_sq = jnp.sum(a * a, axis=1, keepdims=True)
    d_an = jnp.sqrt(jnp.clip(
        a_sq + jnp.min(minacc_ref[...], axis=1, keepdims=True), 1e-12))
    d_ap = jnp.sqrt(jnp.clip(
        a_sq + jnp.sum(posacc_ref[...], axis=1, keepdims=True), 1e-12))
    per = jnp.maximum(d_ap - d_an + MARGIN_, 0.0)
    out_ref[0, 0] = jnp.sum(per) / a.shape[0]


def kernel(inputs, labels, target):
    B, D = inputs.shape
    C = target.shape[0]
    nblocks = (C + CB_ - 1) // CB_

    a_aug = jnp.concatenate(
        [-2.0 * inputs,
         jnp.ones((B, 1), jnp.float32),
         jnp.zeros((B, KAUG_ - D - 1), jnp.float32)], axis=1)
    labels2 = labels.reshape(B, 1)

    minacc, posacc = pl.pallas_call(
        functools.partial(_dist_body, n_valid=C),
        grid=(nblocks,),
        in_specs=[
            pl.BlockSpec((B, KAUG_), lambda i: (0, 0)),
            pl.BlockSpec((B, 1), lambda i: (0, 0)),
            pl.BlockSpec((CB_, D), lambda i: (i, 0)),
        ],
        out_specs=[
            pl.BlockSpec((B, 128), lambda i: (0, 0)),
            pl.BlockSpec((B, 128), lambda i: (0, 0)),
        ],
        out_shape=[
            jax.ShapeDtypeStruct((B, 128), jnp.float32),
            jax.ShapeDtypeStruct((B, 128), jnp.float32),
        ],
        compiler_params=pltpu.CompilerParams(
            dimension_semantics=("arbitrary",)),
    )(a_aug, labels2, target)

    out = pl.pallas_call(
        _final_body,
        out_specs=pl.BlockSpec(memory_space=pltpu.SMEM),
        out_shape=jax.ShapeDtypeStruct((1, 1), jnp.float32),
    )(minacc, posacc, inputs)
    return out[0, 0]
